# Initial kernel scaffold; baseline (speedup 1.0000x reference)
import functools

import jax
import jax.numpy as jnp
from jax import lax
from jax.experimental import pallas as pl
from jax.experimental.pallas import tpu as pltpu
from jax.experimental.pallas import tpu_sc as plsc

NC = 2
NS = 16
K = 80


def _sc_agg(x, src4, dst4, n_nodes, d, nchunk):
    rows_per_tile = n_nodes // NS
    zr = rows_per_tile // 5
    mesh = plsc.VectorSubcoreMesh(
        core_axis_name="c", subcore_axis_name="s", num_cores=NC, num_subcores=NS
    )

    @functools.partial(
        pl.kernel,
        out_type=jax.ShapeDtypeStruct((NC, n_nodes, d), jnp.float32),
        mesh=mesh,
        scratch_types=[
            pltpu.VMEM((nchunk, K), jnp.int32),
            pltpu.VMEM((nchunk, K), jnp.int32),
            pltpu.VMEM((K, d), jnp.float32),
            pltpu.VMEM((zr, d), jnp.float32),
            pltpu.VMEM_SHARED((n_nodes, d), jnp.float32),
        ],
    )
    def agg_kernel(x_hbm, src_hbm, dst_hbm, out_hbm, src_v, dst_v, rows_v,
                   zero_v, acc_sh):
        c = lax.axis_index("c")
        s = lax.axis_index("s")
        pltpu.sync_copy(src_hbm.at[c, s], src_v)
        pltpu.sync_copy(dst_hbm.at[c, s], dst_v)

        @pl.loop(0, zr)
        def _(r):
            @pl.loop(0, d, step=16)
            def _(o):
                zero_v[r, pl.ds(o, 16)] = jnp.zeros((16,), jnp.float32)

        row0 = s * rows_per_tile

        @pl.loop(0, 5)
        def _(z):
            pltpu.sync_copy(zero_v, acc_sh.at[pl.ds(row0 + z * zr, zr)])

        plsc.subcore_barrier()

        @pl.loop(0, nchunk)
        def _(j):
            pltpu.sync_copy(x_hbm.at[src_v.at[j]], rows_v)
            pltpu.sync_copy(rows_v, acc_sh.at[dst_v.at[j]], add=True)

        plsc.subcore_barrier()
        pltpu.sync_copy(acc_sh.at[pl.ds(row0, rows_per_tile)],
                        out_hbm.at[c, pl.ds(row0, rows_per_tile)])

    return agg_kernel(x, src4, dst4)


def _sc_count(dst4, n_nodes, nchunk):
    rows_per_tile = n_nodes // NS
    zr = rows_per_tile // 5
    mesh = plsc.VectorSubcoreMesh(
        core_axis_name="c", subcore_axis_name="s", num_cores=NC, num_subcores=NS
    )

    @functools.partial(
        pl.kernel,
        out_type=jax.ShapeDtypeStruct((NC, n_nodes, 16), jnp.float32),
        mesh=mesh,
        scratch_types=[
            pltpu.VMEM((nchunk, K), jnp.int32),
            pltpu.VMEM((K, 16), jnp.float32),
            pltpu.VMEM((zr, 16), jnp.float32),
            pltpu.VMEM_SHARED((n_nodes, 16), jnp.float32),
        ],
    )
    def count_kernel(dst_hbm, out_hbm, dst_v, ones_v, zero_v, cnt_sh):
        c = lax.axis_index("c")
        s = lax.axis_index("s")
        pltpu.sync_copy(dst_hbm.at[c, s], dst_v)

        @pl.loop(0, K)
        def _(r):
            ones_v[r, pl.ds(0, 16)] = jnp.full((16,), 1.0, jnp.float32)

        @pl.loop(0, zr)
        def _(r):
            zero_v[r, pl.ds(0, 16)] = jnp.zeros((16,), jnp.float32)

        row0 = s * rows_per_tile

        @pl.loop(0, 5)
        def _(z):
            pltpu.sync_copy(zero_v, cnt_sh.at[pl.ds(row0 + z * zr, zr)])

        plsc.subcore_barrier()

        @pl.loop(0, nchunk)
        def _(j):
            pltpu.sync_copy(ones_v, cnt_sh.at[dst_v.at[j]], add=True)

        plsc.subcore_barrier()
        pltpu.sync_copy(cnt_sh.at[pl.ds(row0, rows_per_tile)],
                        out_hbm.at[c, pl.ds(row0, rows_per_tile)])

    return count_kernel(dst4)


def _tc_layer(x, parts, cnts, wx, wn, b, n_nodes, d):
    blk = 1000

    def body(x_ref, p_ref, c_ref, wx_ref, wn_ref, b_ref, o_ref):
        cnt = c_ref[0, :, 0:1] + c_ref[1, :, 0:1] + 1.0
        neigh = (p_ref[0] + p_ref[1]) / cnt
        acc = jnp.dot(x_ref[...], wx_ref[...], preferred_element_type=jnp.float32)
        acc = acc + jnp.dot(neigh, wn_ref[...], preferred_element_type=jnp.float32)
        o_ref[...] = jnp.maximum(acc + b_ref[...], 0.0)

    return pl.pallas_call(
        body,
        grid=(n_nodes // blk,),
        in_specs=[
            pl.BlockSpec((blk, d), lambda i: (i, 0)),
            pl.BlockSpec((NC, blk, d), lambda i: (0, i, 0)),
            pl.BlockSpec((NC, blk, 16), lambda i: (0, i, 0)),
            pl.BlockSpec((d, d), lambda i: (0, 0)),
            pl.BlockSpec((d, d), lambda i: (0, 0)),
            pl.BlockSpec((1, d), lambda i: (0, 0)),
        ],
        out_specs=pl.BlockSpec((blk, d), lambda i: (i, 0)),
        out_shape=jax.ShapeDtypeStruct((n_nodes, d), jnp.float32),
    )(x, parts, cnts, wx, wn, b.reshape(1, d))


def kernel(x, edge_index, W0, b0, W1, b1, W2, b2):
    n_nodes, d = x.shape
    n_edges = edge_index.shape[1]
    nchunk = n_edges // (NC * NS * K)
    src4 = edge_index[0].astype(jnp.int32).reshape(NC, NS, nchunk, K)
    dst4 = edge_index[1].astype(jnp.int32).reshape(NC, NS, nchunk, K)

    cnts = _sc_count(dst4, n_nodes, nchunk)
    for (w, b) in ((W0, b0), (W1, b1), (W2, b2)):
        parts = _sc_agg(x, src4, dst4, n_nodes, d, nchunk)
        x = _tc_layer(x, parts, cnts, w[:d], w[d:], b, n_nodes, d)
    return x


# speedup vs baseline: 6.3927x; 6.3927x over previous
import functools

import jax
import jax.numpy as jnp
from jax import lax
from jax.experimental import pallas as pl
from jax.experimental.pallas import tpu as pltpu
from jax.experimental.pallas import tpu_sc as plsc

NC = 2
NS = 16
K = 80


def _sc_agg(x, src4, dst4, n_nodes, d, nchunk):
    nrow = 8 * ((n_nodes // NS) // 8)
    nrow_last = n_nodes - (NS - 1) * nrow
    extra = nrow_last - nrow
    mesh = plsc.VectorSubcoreMesh(
        core_axis_name="c", subcore_axis_name="s", num_cores=NC, num_subcores=NS
    )

    @functools.partial(
        pl.kernel,
        out_type=jax.ShapeDtypeStruct((NC, n_nodes, d), jnp.float32),
        mesh=mesh,
        scratch_types=[
            pltpu.VMEM((nchunk, K), jnp.int32),
            pltpu.VMEM((nchunk, K), jnp.int32),
            pltpu.VMEM((K, d), jnp.float32),
            pltpu.VMEM((16, d), jnp.float32),
            pltpu.VMEM_SHARED((n_nodes, d), jnp.float32),
        ],
    )
    def agg_kernel(x_hbm, src_hbm, dst_hbm, out_hbm, src_v, dst_v, rows_v,
                   zero_v, acc_sh):
        c = lax.axis_index("c")
        s = lax.axis_index("s")
        pltpu.sync_copy(src_hbm.at[c, s], src_v)
        pltpu.sync_copy(dst_hbm.at[c, s], dst_v)

        @pl.loop(0, 16)
        def _(r):
            @pl.loop(0, d, step=16)
            def _(o):
                zero_v[r, pl.ds(o, 16)] = jnp.zeros((16,), jnp.float32)

        row0 = s * nrow

        @pl.loop(0, nrow, step=16)
        def _(z):
            pltpu.sync_copy(zero_v, acc_sh.at[pl.ds(row0 + z, 16)])

        @pl.when(s == NS - 1)
        def _():
            pltpu.sync_copy(zero_v.at[pl.ds(0, extra)],
                            acc_sh.at[pl.ds(row0 + nrow, extra)])

        plsc.subcore_barrier()

        @pl.loop(0, nchunk)
        def _(j):
            pltpu.sync_copy(x_hbm.at[src_v.at[j]], rows_v)
            pltpu.sync_copy(rows_v, acc_sh.at[dst_v.at[j]], add=True)

        plsc.subcore_barrier()
        pltpu.sync_copy(acc_sh.at[pl.ds(row0, nrow)],
                        out_hbm.at[c, pl.ds(row0, nrow)])

        @pl.when(s == NS - 1)
        def _():
            pltpu.sync_copy(acc_sh.at[pl.ds(row0 + nrow, extra)],
                            out_hbm.at[c, pl.ds(row0 + nrow, extra)])

    return agg_kernel(x, src4, dst4)


def _sc_count(dst4, n_nodes, d, nchunk):
    nrow = 8 * ((n_nodes // NS) // 8)
    nrow_last = n_nodes - (NS - 1) * nrow
    extra = nrow_last - nrow
    mesh = plsc.VectorSubcoreMesh(
        core_axis_name="c", subcore_axis_name="s", num_cores=NC, num_subcores=NS
    )

    @functools.partial(
        pl.kernel,
        out_type=jax.ShapeDtypeStruct((NC, n_nodes, d), jnp.float32),
        mesh=mesh,
        scratch_types=[
            pltpu.VMEM((nchunk, K), jnp.int32),
            pltpu.VMEM((K, d), jnp.float32),
            pltpu.VMEM((16, d), jnp.float32),
            pltpu.VMEM_SHARED((n_nodes, d), jnp.float32),
        ],
    )
    def count_kernel(dst_hbm, out_hbm, dst_v, ones_v, zero_v, cnt_sh):
        c = lax.axis_index("c")
        s = lax.axis_index("s")
        pltpu.sync_copy(dst_hbm.at[c, s], dst_v)

        @pl.loop(0, K)
        def _(r):
            @pl.loop(0, d, step=16)
            def _(o):
                ones_v[r, pl.ds(o, 16)] = jnp.full((16,), 1.0, jnp.float32)

        @pl.loop(0, 16)
        def _(r):
            @pl.loop(0, d, step=16)
            def _(o):
                zero_v[r, pl.ds(o, 16)] = jnp.zeros((16,), jnp.float32)

        row0 = s * nrow

        @pl.loop(0, nrow, step=16)
        def _(z):
            pltpu.sync_copy(zero_v, cnt_sh.at[pl.ds(row0 + z, 16)])

        @pl.when(s == NS - 1)
        def _():
            pltpu.sync_copy(zero_v.at[pl.ds(0, extra)],
                            cnt_sh.at[pl.ds(row0 + nrow, extra)])

        plsc.subcore_barrier()

        @pl.loop(0, nchunk)
        def _(j):
            pltpu.sync_copy(ones_v, cnt_sh.at[dst_v.at[j]], add=True)

        plsc.subcore_barrier()
        pltpu.sync_copy(cnt_sh.at[pl.ds(row0, nrow)],
                        out_hbm.at[c, pl.ds(row0, nrow)])

        @pl.when(s == NS - 1)
        def _():
            pltpu.sync_copy(cnt_sh.at[pl.ds(row0 + nrow, extra)],
                            out_hbm.at[c, pl.ds(row0 + nrow, extra)])

    return count_kernel(dst4)


def _tc_layer(x, parts, cnts, wx, wn, b, n_nodes, d):
    blk = 1000

    def body(x_ref, p_ref, c_ref, wx_ref, wn_ref, b_ref, o_ref):
        cnt = c_ref[0, :, 0:1] + c_ref[1, :, 0:1] + 1.0
        neigh = (p_ref[0] + p_ref[1]) / cnt
        acc = jnp.dot(x_ref[...], wx_ref[...], preferred_element_type=jnp.float32)
        acc = acc + jnp.dot(neigh, wn_ref[...], preferred_element_type=jnp.float32)
        o_ref[...] = jnp.maximum(acc + b_ref[...], 0.0)

    return pl.pallas_call(
        body,
        grid=(n_nodes // blk,),
        in_specs=[
            pl.BlockSpec((blk, d), lambda i: (i, 0)),
            pl.BlockSpec((NC, blk, d), lambda i: (0, i, 0)),
            pl.BlockSpec((NC, blk, d), lambda i: (0, i, 0)),
            pl.BlockSpec((d, d), lambda i: (0, 0)),
            pl.BlockSpec((d, d), lambda i: (0, 0)),
            pl.BlockSpec((1, d), lambda i: (0, 0)),
        ],
        out_specs=pl.BlockSpec((blk, d), lambda i: (i, 0)),
        out_shape=jax.ShapeDtypeStruct((n_nodes, d), jnp.float32),
    )(x, parts, cnts, wx, wn, b.reshape(1, d))


def kernel(x, edge_index, W0, b0, W1, b1, W2, b2):
    n_nodes, d = x.shape
    n_edges = edge_index.shape[1]
    nchunk = n_edges // (NC * NS * K)
    src4 = edge_index[0].astype(jnp.int32).reshape(NC, NS, nchunk, K)
    dst4 = edge_index[1].astype(jnp.int32).reshape(NC, NS, nchunk, K)

    cnts = _sc_count(dst4, n_nodes, d, nchunk)
    for (w, b) in ((W0, b0), (W1, b1), (W2, b2)):
        parts = _sc_agg(x, src4, dst4, n_nodes, d, nchunk)
        x = _tc_layer(x, parts, cnts, w[:d], w[d:], b, n_nodes, d)
    return x


# speedup vs baseline: 8.4616x; 1.3236x over previous
import functools

import jax
import jax.numpy as jnp
from jax import lax
from jax.experimental import pallas as pl
from jax.experimental.pallas import tpu as pltpu
from jax.experimental.pallas import tpu_sc as plsc

NC = 2
NS = 16
K = 128
IB = 8
SINK = 16


def _sc_agg(x, src4, dst4, n_nodes, d, nchunk):
    nrow = 8 * ((n_nodes // NS) // 8)
    extra = n_nodes - NS * nrow
    nblk = nchunk // IB
    mesh = plsc.VectorSubcoreMesh(
        core_axis_name="c", subcore_axis_name="s", num_cores=NC, num_subcores=NS
    )

    @functools.partial(
        pl.kernel,
        out_type=jax.ShapeDtypeStruct((NC, n_nodes, d), jnp.float32),
        mesh=mesh,
        scratch_types=[
            pltpu.VMEM((nchunk, K), jnp.int32),
            pltpu.VMEM((IB, K), jnp.int32),
            pltpu.VMEM((IB, K), jnp.int32),
            pltpu.VMEM((K, d), jnp.float32),
            pltpu.VMEM((K, d), jnp.float32),
            pltpu.VMEM((16, d), jnp.float32),
            pltpu.VMEM_SHARED((n_nodes + SINK, d), jnp.float32),
            pltpu.SemaphoreType.DMA,
            pltpu.SemaphoreType.DMA,
            pltpu.SemaphoreType.DMA,
            pltpu.SemaphoreType.DMA,
        ],
    )
    def agg_kernel(x_hbm, src_hbm, dst_hbm, out_hbm, dst_v, ia_v, ib_v,
                   r0_v, r1_v, zero_v, acc_sh, g0_sem, g1_sem, ia_sem, ib_sem):
        c = lax.axis_index("c")
        s = lax.axis_index("s")
        pltpu.sync_copy(dst_hbm.at[c, s], dst_v)

        @pl.loop(0, 16)
        def _(r):
            @pl.loop(0, d, step=16)
            def _(o):
                zero_v[r, pl.ds(o, 16)] = jnp.zeros((16,), jnp.float32)

        row0 = s * nrow

        @pl.loop(0, nrow, step=16)
        def _(z):
            pltpu.sync_copy(zero_v, acc_sh.at[pl.ds(row0 + z, 16)])

        @pl.when(s == NS - 1)
        def _():
            @pl.loop(0, extra + SINK, step=16)
            def _(z):
                pltpu.sync_copy(zero_v, acc_sh.at[pl.ds(row0 + nrow + z, 16)])

        plsc.subcore_barrier()

        rbufs = (r0_v, r1_v)
        gsems = (g0_sem, g1_sem)

        def idx_src(b, buf, sem):
            return pltpu.make_async_copy(
                src_hbm.at[c, s, pl.ds(b * IB, IB)], buf, sem)

        def gather(idx_row, k):
            return pltpu.make_async_copy(
                x_hbm.at[idx_row], rbufs[k % 2], gsems[k % 2])

        pltpu.sync_copy(src_hbm.at[c, s, pl.ds(0, IB)], ia_v)
        idx_src(1, ib_v, ib_sem).start()
        gather(ia_v.at[0], 0).start()

        def block(b, cur, nxt, cur_sem, nxt_sem):
            for k in range(IB):
                j = b * IB + k
                gather(cur.at[k], k).wait()
                if k < IB - 1:
                    gather(cur.at[k + 1], k + 1).start()
                else:
                    @pl.when(b + 2 < nblk)
                    def _():
                        idx_src(b + 2, cur, cur_sem).start()

                    @pl.when(b + 1 < nblk)
                    def _():
                        idx_src(b + 1, nxt, nxt_sem).wait()
                        gather(nxt.at[0], 0).start()
                pltpu.sync_copy(rbufs[k % 2], acc_sh.at[dst_v.at[j]], add=True)

        @pl.loop(0, nblk, step=2)
        def _(b):
            block(b, ia_v, ib_v, ia_sem, ib_sem)
            block(b + 1, ib_v, ia_v, ib_sem, ia_sem)

        plsc.subcore_barrier()
        pltpu.sync_copy(acc_sh.at[pl.ds(row0, nrow)],
                        out_hbm.at[c, pl.ds(row0, nrow)])

        @pl.when(s == NS - 1)
        def _():
            pltpu.sync_copy(acc_sh.at[pl.ds(row0 + nrow, extra)],
                            out_hbm.at[c, pl.ds(row0 + nrow, extra)])

    return agg_kernel(x, src4, dst4)


def _sc_count(dst4, n_nodes, d, nchunk):
    nrow = 8 * ((n_nodes // NS) // 8)
    extra = n_nodes - NS * nrow
    mesh = plsc.VectorSubcoreMesh(
        core_axis_name="c", subcore_axis_name="s", num_cores=NC, num_subcores=NS
    )

    @functools.partial(
        pl.kernel,
        out_type=jax.ShapeDtypeStruct((NC, n_nodes, d), jnp.float32),
        mesh=mesh,
        scratch_types=[
            pltpu.VMEM((nchunk, K), jnp.int32),
            pltpu.VMEM((K, d), jnp.float32),
            pltpu.VMEM((16, d), jnp.float32),
            pltpu.VMEM_SHARED((n_nodes + SINK, d), jnp.float32),
        ],
    )
    def count_kernel(dst_hbm, out_hbm, dst_v, ones_v, zero_v, cnt_sh):
        c = lax.axis_index("c")
        s = lax.axis_index("s")
        pltpu.sync_copy(dst_hbm.at[c, s], dst_v)

        @pl.loop(0, K)
        def _(r):
            @pl.loop(0, d, step=16)
            def _(o):
                ones_v[r, pl.ds(o, 16)] = jnp.full((16,), 1.0, jnp.float32)

        @pl.loop(0, 16)
        def _(r):
            @pl.loop(0, d, step=16)
            def _(o):
                zero_v[r, pl.ds(o, 16)] = jnp.zeros((16,), jnp.float32)

        row0 = s * nrow

        @pl.loop(0, nrow, step=16)
        def _(z):
            pltpu.sync_copy(zero_v, cnt_sh.at[pl.ds(row0 + z, 16)])

        @pl.when(s == NS - 1)
        def _():
            @pl.loop(0, extra + SINK, step=16)
            def _(z):
                pltpu.sync_copy(zero_v, cnt_sh.at[pl.ds(row0 + nrow + z, 16)])

        plsc.subcore_barrier()

        @pl.loop(0, nchunk)
        def _(j):
            pltpu.sync_copy(ones_v, cnt_sh.at[dst_v.at[j]], add=True)

        plsc.subcore_barrier()
        pltpu.sync_copy(cnt_sh.at[pl.ds(row0, nrow)],
                        out_hbm.at[c, pl.ds(row0, nrow)])

        @pl.when(s == NS - 1)
        def _():
            pltpu.sync_copy(cnt_sh.at[pl.ds(row0 + nrow, extra)],
                            out_hbm.at[c, pl.ds(row0 + nrow, extra)])

    return count_kernel(dst4)


def _tc_layer(x, parts, cnts, wx, wn, b, n_nodes, d):
    blk = 1000

    def body(x_ref, p_ref, c_ref, wx_ref, wn_ref, b_ref, o_ref):
        cnt = c_ref[0, :, 0:1] + c_ref[1, :, 0:1] + 1.0
        neigh = (p_ref[0] + p_ref[1]) / cnt
        acc = jnp.dot(x_ref[...], wx_ref[...], preferred_element_type=jnp.float32)
        acc = acc + jnp.dot(neigh, wn_ref[...], preferred_element_type=jnp.float32)
        o_ref[...] = jnp.maximum(acc + b_ref[...], 0.0)

    return pl.pallas_call(
        body,
        grid=(n_nodes // blk,),
        in_specs=[
            pl.BlockSpec((blk, d), lambda i: (i, 0)),
            pl.BlockSpec((NC, blk, d), lambda i: (0, i, 0)),
            pl.BlockSpec((NC, blk, d), lambda i: (0, i, 0)),
            pl.BlockSpec((d, d), lambda i: (0, 0)),
            pl.BlockSpec((d, d), lambda i: (0, 0)),
            pl.BlockSpec((1, d), lambda i: (0, 0)),
        ],
        out_specs=pl.BlockSpec((blk, d), lambda i: (i, 0)),
        out_shape=jax.ShapeDtypeStruct((n_nodes, d), jnp.float32),
    )(x, parts, cnts, wx, wn, b.reshape(1, d))


def kernel(x, edge_index, W0, b0, W1, b1, W2, b2):
    n_nodes, d = x.shape
    n_edges = edge_index.shape[1]
    e_tile = n_edges // (NC * NS)
    nchunk = -(-e_tile // (K * IB)) * IB
    pad = nchunk * K - e_tile

    lane = jnp.arange(pad, dtype=jnp.int32) % SINK
    src2 = edge_index[0].astype(jnp.int32).reshape(NC * NS, e_tile)
    dst2 = edge_index[1].astype(jnp.int32).reshape(NC * NS, e_tile)
    src_pad = jnp.broadcast_to(lane, (NC * NS, pad))
    dst_pad = jnp.broadcast_to(n_nodes + lane, (NC * NS, pad))
    src4 = jnp.concatenate([src2, src_pad], 1).reshape(NC, NS, nchunk, K)
    dst4 = jnp.concatenate([dst2, dst_pad], 1).reshape(NC, NS, nchunk, K)

    cnts = _sc_count(dst4, n_nodes, d, nchunk)
    for (w, b) in ((W0, b0), (W1, b1), (W2, b2)):
        parts = _sc_agg(x, src4, dst4, n_nodes, d, nchunk)
        x = _tc_layer(x, parts, cnts, w[:d], w[d:], b, n_nodes, d)
    return x


# speedup vs baseline: 10.2167x; 1.2074x over previous
import functools

import jax
import jax.numpy as jnp
from jax import lax
from jax.experimental import pallas as pl
from jax.experimental.pallas import tpu as pltpu
from jax.experimental.pallas import tpu_sc as plsc

NC = 2
NS = 16
K = 64
IB = 8
NB = 4
SINK = 16


def _sc_agg(x, src4, dst4, n_nodes, d, nchunk):
    nrow = 8 * ((n_nodes // NS) // 8)
    extra = n_nodes - NS * nrow
    nblk = nchunk // IB
    mesh = plsc.VectorSubcoreMesh(
        core_axis_name="c", subcore_axis_name="s", num_cores=NC, num_subcores=NS
    )

    @functools.partial(
        pl.kernel,
        out_type=jax.ShapeDtypeStruct((NC, n_nodes, d), jnp.float32),
        mesh=mesh,
        scratch_types=[
            pltpu.VMEM((IB, K), jnp.int32),
            pltpu.VMEM((IB, K), jnp.int32),
            pltpu.VMEM((IB, K), jnp.int32),
            pltpu.VMEM((IB, K), jnp.int32),
            pltpu.VMEM((NB, K, d), jnp.float32),
            pltpu.VMEM((16, d), jnp.float32),
            pltpu.VMEM_SHARED((n_nodes + SINK, d), jnp.float32),
            pltpu.SemaphoreType.DMA,
            pltpu.SemaphoreType.DMA,
            pltpu.SemaphoreType.DMA,
            pltpu.SemaphoreType.DMA,
            pltpu.SemaphoreType.DMA,
            pltpu.SemaphoreType.DMA,
            pltpu.SemaphoreType.DMA,
            pltpu.SemaphoreType.DMA,
        ],
    )
    def agg_kernel(x_hbm, src_hbm, dst_hbm, out_hbm, sa_v, sb_v, da_v, db_v,
                   rows_v, zero_v, acc_sh, g0, g1, g2, g3, sa_sem, sb_sem,
                   da_sem, db_sem):
        c = lax.axis_index("c")
        s = lax.axis_index("s")

        @pl.loop(0, 16)
        def _(r):
            @pl.loop(0, d, step=16)
            def _(o):
                zero_v[r, pl.ds(o, 16)] = jnp.zeros((16,), jnp.float32)

        row0 = s * nrow

        @pl.loop(0, nrow, step=16)
        def _(z):
            pltpu.sync_copy(zero_v, acc_sh.at[pl.ds(row0 + z, 16)])

        @pl.when(s == NS - 1)
        def _():
            @pl.loop(0, extra + SINK, step=16)
            def _(z):
                pltpu.sync_copy(zero_v, acc_sh.at[pl.ds(row0 + nrow + z, 16)])

        plsc.subcore_barrier()

        gsems = (g0, g1, g2, g3)

        def src_blk(b, buf, sem):
            return pltpu.make_async_copy(
                src_hbm.at[c, s, pl.ds(b * IB, IB)], buf, sem)

        def dst_blk(b, buf, sem):
            return pltpu.make_async_copy(
                dst_hbm.at[c, s, pl.ds(b * IB, IB)], buf, sem)

        def gather(idx_row, j_mod):
            return pltpu.make_async_copy(
                x_hbm.at[idx_row], rows_v.at[j_mod], gsems[j_mod])

        pltpu.sync_copy(src_hbm.at[c, s, pl.ds(0, IB)], sa_v)
        src_blk(1, sb_v, sb_sem).start()
        dst_blk(0, da_v, da_sem).start()
        dst_blk(1, db_v, db_sem).start()
        gather(sa_v.at[0], 0).start()
        gather(sa_v.at[1], 1).start()
        gather(sa_v.at[2], 2).start()

        def block(b, cur, nxt, curd, nxtd, cur_ssem, nxt_ssem, curd_sem):
            dst_blk(b, curd, curd_sem).wait()
            for k in range(IB):
                gather(cur.at[k], k % NB).wait()
                if k == IB - 3:
                    @pl.when(b + 1 < nblk)
                    def _():
                        src_blk(b + 1, nxt, nxt_ssem).wait()
                if k < IB - 3:
                    gather(cur.at[k + 3], (k + 3) % NB).start()
                else:
                    @pl.when(b * IB + k + 3 < nchunk)
                    def _():
                        gather(nxt.at[k - (IB - 3)], (k + 3) % NB).start()
                if k == IB - 1:
                    @pl.when(b + 2 < nblk)
                    def _():
                        src_blk(b + 2, cur, cur_ssem).start()
                pltpu.sync_copy(rows_v.at[k % NB], acc_sh.at[curd.at[k]],
                                add=True)
            @pl.when(b + 2 < nblk)
            def _():
                dst_blk(b + 2, curd, curd_sem).start()

        @pl.loop(0, nblk, step=2)
        def _(b):
            block(b, sa_v, sb_v, da_v, db_v, sa_sem, sb_sem, da_sem)
            block(b + 1, sb_v, sa_v, db_v, da_v, sb_sem, sa_sem, db_sem)

        plsc.subcore_barrier()
        pltpu.sync_copy(acc_sh.at[pl.ds(row0, nrow)],
                        out_hbm.at[c, pl.ds(row0, nrow)])

        @pl.when(s == NS - 1)
        def _():
            pltpu.sync_copy(acc_sh.at[pl.ds(row0 + nrow, extra)],
                            out_hbm.at[c, pl.ds(row0 + nrow, extra)])

    return agg_kernel(x, src4, dst4)


def _sc_count(dst4, n_nodes, d, nchunk):
    nrow = 8 * ((n_nodes // NS) // 8)
    extra = n_nodes - NS * nrow
    mesh = plsc.VectorSubcoreMesh(
        core_axis_name="c", subcore_axis_name="s", num_cores=NC, num_subcores=NS
    )

    @functools.partial(
        pl.kernel,
        out_type=jax.ShapeDtypeStruct((NC, n_nodes, d), jnp.float32),
        mesh=mesh,
        scratch_types=[
            pltpu.VMEM((nchunk, K), jnp.int32),
            pltpu.VMEM((K, d), jnp.float32),
            pltpu.VMEM((16, d), jnp.float32),
            pltpu.VMEM_SHARED((n_nodes + SINK, d), jnp.float32),
        ],
    )
    def count_kernel(dst_hbm, out_hbm, dst_v, ones_v, zero_v, cnt_sh):
        c = lax.axis_index("c")
        s = lax.axis_index("s")
        pltpu.sync_copy(dst_hbm.at[c, s], dst_v)

        @pl.loop(0, K)
        def _(r):
            @pl.loop(0, d, step=16)
            def _(o):
                ones_v[r, pl.ds(o, 16)] = jnp.full((16,), 1.0, jnp.float32)

        @pl.loop(0, 16)
        def _(r):
            @pl.loop(0, d, step=16)
            def _(o):
                zero_v[r, pl.ds(o, 16)] = jnp.zeros((16,), jnp.float32)

        row0 = s * nrow

        @pl.loop(0, nrow, step=16)
        def _(z):
            pltpu.sync_copy(zero_v, cnt_sh.at[pl.ds(row0 + z, 16)])

        @pl.when(s == NS - 1)
        def _():
            @pl.loop(0, extra + SINK, step=16)
            def _(z):
                pltpu.sync_copy(zero_v, cnt_sh.at[pl.ds(row0 + nrow + z, 16)])

        plsc.subcore_barrier()

        @pl.loop(0, nchunk)
        def _(j):
            pltpu.sync_copy(ones_v, cnt_sh.at[dst_v.at[j]], add=True)

        plsc.subcore_barrier()
        pltpu.sync_copy(cnt_sh.at[pl.ds(row0, nrow)],
                        out_hbm.at[c, pl.ds(row0, nrow)])

        @pl.when(s == NS - 1)
        def _():
            pltpu.sync_copy(cnt_sh.at[pl.ds(row0 + nrow, extra)],
                            out_hbm.at[c, pl.ds(row0 + nrow, extra)])

    return count_kernel(dst4)


def _tc_layer(x, parts, cnts, wx, wn, b, n_nodes, d):
    blk = 1000

    def body(x_ref, p_ref, c_ref, wx_ref, wn_ref, b_ref, o_ref):
        cnt = c_ref[0, :, 0:1] + c_ref[1, :, 0:1] + 1.0
        neigh = (p_ref[0] + p_ref[1]) / cnt
        acc = jnp.dot(x_ref[...], wx_ref[...], preferred_element_type=jnp.float32)
        acc = acc + jnp.dot(neigh, wn_ref[...], preferred_element_type=jnp.float32)
        o_ref[...] = jnp.maximum(acc + b_ref[...], 0.0)

    return pl.pallas_call(
        body,
        grid=(n_nodes // blk,),
        in_specs=[
            pl.BlockSpec((blk, d), lambda i: (i, 0)),
            pl.BlockSpec((NC, blk, d), lambda i: (0, i, 0)),
            pl.BlockSpec((NC, blk, d), lambda i: (0, i, 0)),
            pl.BlockSpec((d, d), lambda i: (0, 0)),
            pl.BlockSpec((d, d), lambda i: (0, 0)),
            pl.BlockSpec((1, d), lambda i: (0, 0)),
        ],
        out_specs=pl.BlockSpec((blk, d), lambda i: (i, 0)),
        out_shape=jax.ShapeDtypeStruct((n_nodes, d), jnp.float32),
    )(x, parts, cnts, wx, wn, b.reshape(1, d))


def kernel(x, edge_index, W0, b0, W1, b1, W2, b2):
    n_nodes, d = x.shape
    n_edges = edge_index.shape[1]
    e_tile = n_edges // (NC * NS)
    nchunk = -(-e_tile // (K * IB)) * IB
    pad = nchunk * K - e_tile

    lane = jnp.arange(pad, dtype=jnp.int32) % SINK
    src2 = edge_index[0].astype(jnp.int32).reshape(NC * NS, e_tile)
    dst2 = edge_index[1].astype(jnp.int32).reshape(NC * NS, e_tile)
    src_pad = jnp.broadcast_to(lane, (NC * NS, pad))
    dst_pad = jnp.broadcast_to(n_nodes + lane, (NC * NS, pad))
    src4 = jnp.concatenate([src2, src_pad], 1).reshape(NC, NS, nchunk, K)
    dst4 = jnp.concatenate([dst2, dst_pad], 1).reshape(NC, NS, nchunk, K)

    cnts = _sc_count(dst4, n_nodes, d, nchunk)
    for (w, b) in ((W0, b0), (W1, b1), (W2, b2)):
        parts = _sc_agg(x, src4, dst4, n_nodes, d, nchunk)
        x = _tc_layer(x, parts, cnts, w[:d], w[d:], b, n_nodes, d)
    return x


# speedup vs baseline: 10.2364x; 1.0019x over previous
import functools

import jax
import jax.numpy as jnp
from jax import lax
from jax.experimental import pallas as pl
from jax.experimental.pallas import tpu as pltpu
from jax.experimental.pallas import tpu_sc as plsc

NC = 2
NS = 16
K = 64
IB = 8
NB = 4
SINK = 16


def _sc_agg(x, src4, dst4, n_nodes, d, nchunk):
    nrow = 8 * ((n_nodes // NS) // 8)
    extra = n_nodes - NS * nrow
    nblk = nchunk // IB
    mesh = plsc.VectorSubcoreMesh(
        core_axis_name="c", subcore_axis_name="s", num_cores=NC, num_subcores=NS
    )

    @functools.partial(
        pl.kernel,
        out_type=jax.ShapeDtypeStruct((NC, n_nodes, d), jnp.float32),
        mesh=mesh,
        scratch_types=[
            pltpu.VMEM((IB, K), jnp.int32),
            pltpu.VMEM((IB, K), jnp.int32),
            pltpu.VMEM((IB, K), jnp.int32),
            pltpu.VMEM((IB, K), jnp.int32),
            pltpu.VMEM((NB, K, d), jnp.float32),
            pltpu.VMEM((16, d), jnp.float32),
            pltpu.VMEM_SHARED((n_nodes + SINK, d), jnp.float32),
            pltpu.SemaphoreType.DMA,
            pltpu.SemaphoreType.DMA,
            pltpu.SemaphoreType.DMA,
            pltpu.SemaphoreType.DMA,
            pltpu.SemaphoreType.DMA,
            pltpu.SemaphoreType.DMA,
            pltpu.SemaphoreType.DMA,
            pltpu.SemaphoreType.DMA,
            pltpu.SemaphoreType.DMA,
            pltpu.SemaphoreType.DMA,
            pltpu.SemaphoreType.DMA,
            pltpu.SemaphoreType.DMA,
        ],
    )
    def agg_kernel(x_hbm, src_hbm, dst_hbm, out_hbm, sa_v, sb_v, da_v, db_v,
                   rows_v, zero_v, acc_sh, gl0, gl1, gl2, gl3, gh0, gh1, gh2,
                   gh3, sa_sem, sb_sem, da_sem, db_sem):
        c = lax.axis_index("c")
        s = lax.axis_index("s")

        @pl.loop(0, 16)
        def _(r):
            @pl.loop(0, d, step=16)
            def _(o):
                zero_v[r, pl.ds(o, 16)] = jnp.zeros((16,), jnp.float32)

        row0 = s * nrow

        @pl.loop(0, nrow, step=16)
        def _(z):
            pltpu.sync_copy(zero_v, acc_sh.at[pl.ds(row0 + z, 16)])

        @pl.when(s == NS - 1)
        def _():
            @pl.loop(0, extra + SINK, step=16)
            def _(z):
                pltpu.sync_copy(zero_v, acc_sh.at[pl.ds(row0 + nrow + z, 16)])

        plsc.subcore_barrier()

        glo = (gl0, gl1, gl2, gl3)
        ghi = (gh0, gh1, gh2, gh3)
        H = K // 2

        def src_blk(b, buf, sem):
            return pltpu.make_async_copy(
                src_hbm.at[c, s, pl.ds(b * IB, IB)], buf, sem)

        def dst_blk(b, buf, sem):
            return pltpu.make_async_copy(
                dst_hbm.at[c, s, pl.ds(b * IB, IB)], buf, sem)

        def gather_lo(idx, k, slot):
            return pltpu.make_async_copy(
                x_hbm.at[idx.at[k, pl.ds(0, H)]],
                rows_v.at[slot, pl.ds(0, H)], glo[slot])

        def gather_hi(idx, k, slot):
            return pltpu.make_async_copy(
                x_hbm.at[idx.at[k, pl.ds(H, H)]],
                rows_v.at[slot, pl.ds(H, H)], ghi[slot])

        def gather_start(idx, k, slot):
            gather_lo(idx, k, slot).start()
            gather_hi(idx, k, slot).start()

        def gather_wait(idx, k, slot):
            gather_lo(idx, k, slot).wait()
            gather_hi(idx, k, slot).wait()

        pltpu.sync_copy(src_hbm.at[c, s, pl.ds(0, IB)], sa_v)
        src_blk(1, sb_v, sb_sem).start()
        dst_blk(0, da_v, da_sem).start()
        dst_blk(1, db_v, db_sem).start()
        gather_start(sa_v, 0, 0)
        gather_start(sa_v, 1, 1)
        gather_start(sa_v, 2, 2)

        def block(b, cur, nxt, curd, nxtd, cur_ssem, nxt_ssem, curd_sem):
            dst_blk(b, curd, curd_sem).wait()
            for k in range(IB):
                gather_wait(cur, k, k % NB)
                if k == IB - 3:
                    @pl.when(b + 1 < nblk)
                    def _():
                        src_blk(b + 1, nxt, nxt_ssem).wait()
                if k < IB - 3:
                    gather_start(cur, k + 3, (k + 3) % NB)
                else:
                    @pl.when(b * IB + k + 3 < nchunk)
                    def _():
                        gather_start(nxt, k - (IB - 3), (k + 3) % NB)
                if k == IB - 1:
                    @pl.when(b + 2 < nblk)
                    def _():
                        src_blk(b + 2, cur, cur_ssem).start()
                pltpu.sync_copy(rows_v.at[k % NB], acc_sh.at[curd.at[k]],
                                add=True)
            @pl.when(b + 2 < nblk)
            def _():
                dst_blk(b + 2, curd, curd_sem).start()

        @pl.loop(0, nblk, step=2)
        def _(b):
            block(b, sa_v, sb_v, da_v, db_v, sa_sem, sb_sem, da_sem)
            block(b + 1, sb_v, sa_v, db_v, da_v, sb_sem, sa_sem, db_sem)

        plsc.subcore_barrier()
        pltpu.sync_copy(acc_sh.at[pl.ds(row0, nrow)],
                        out_hbm.at[c, pl.ds(row0, nrow)])

        @pl.when(s == NS - 1)
        def _():
            pltpu.sync_copy(acc_sh.at[pl.ds(row0 + nrow, extra)],
                            out_hbm.at[c, pl.ds(row0 + nrow, extra)])

    return agg_kernel(x, src4, dst4)


def _sc_count(dst4, n_nodes, d, nchunk):
    nrow = 8 * ((n_nodes // NS) // 8)
    extra = n_nodes - NS * nrow
    mesh = plsc.VectorSubcoreMesh(
        core_axis_name="c", subcore_axis_name="s", num_cores=NC, num_subcores=NS
    )

    @functools.partial(
        pl.kernel,
        out_type=jax.ShapeDtypeStruct((NC, n_nodes, d), jnp.float32),
        mesh=mesh,
        scratch_types=[
            pltpu.VMEM((nchunk, K), jnp.int32),
            pltpu.VMEM((K, d), jnp.float32),
            pltpu.VMEM((16, d), jnp.float32),
            pltpu.VMEM_SHARED((n_nodes + SINK, d), jnp.float32),
        ],
    )
    def count_kernel(dst_hbm, out_hbm, dst_v, ones_v, zero_v, cnt_sh):
        c = lax.axis_index("c")
        s = lax.axis_index("s")
        pltpu.sync_copy(dst_hbm.at[c, s], dst_v)

        @pl.loop(0, K)
        def _(r):
            @pl.loop(0, d, step=16)
            def _(o):
                ones_v[r, pl.ds(o, 16)] = jnp.full((16,), 1.0, jnp.float32)

        @pl.loop(0, 16)
        def _(r):
            @pl.loop(0, d, step=16)
            def _(o):
                zero_v[r, pl.ds(o, 16)] = jnp.zeros((16,), jnp.float32)

        row0 = s * nrow

        @pl.loop(0, nrow, step=16)
        def _(z):
            pltpu.sync_copy(zero_v, cnt_sh.at[pl.ds(row0 + z, 16)])

        @pl.when(s == NS - 1)
        def _():
            @pl.loop(0, extra + SINK, step=16)
            def _(z):
                pltpu.sync_copy(zero_v, cnt_sh.at[pl.ds(row0 + nrow + z, 16)])

        plsc.subcore_barrier()

        @pl.loop(0, nchunk)
        def _(j):
            pltpu.sync_copy(ones_v, cnt_sh.at[dst_v.at[j]], add=True)

        plsc.subcore_barrier()

        pltpu.sync_copy(cnt_sh.at[pl.ds(row0, nrow)],
                        out_hbm.at[c, pl.ds(row0, nrow)])

        @pl.when(s == NS - 1)
        def _():
            pltpu.sync_copy(cnt_sh.at[pl.ds(row0 + nrow, extra)],
                            out_hbm.at[c, pl.ds(row0 + nrow, extra)])

    return count_kernel(dst4)


def _tc_layer(x, parts, cnts, wx, wn, b, n_nodes, d):
    blk = 1000

    def body(x_ref, p_ref, c_ref, wx_ref, wn_ref, b_ref, o_ref):
        cnt = c_ref[0, :, 0:1] + c_ref[1, :, 0:1] + 1.0
        neigh = (p_ref[0] + p_ref[1]) / cnt
        acc = jnp.dot(x_ref[...], wx_ref[...], preferred_element_type=jnp.float32)
        acc = acc + jnp.dot(neigh, wn_ref[...], preferred_element_type=jnp.float32)
        o_ref[...] = jnp.maximum(acc + b_ref[...], 0.0)

    return pl.pallas_call(
        body,
        grid=(n_nodes // blk,),
        in_specs=[
            pl.BlockSpec((blk, d), lambda i: (i, 0)),
            pl.BlockSpec((NC, blk, d), lambda i: (0, i, 0)),
            pl.BlockSpec((NC, blk, d), lambda i: (0, i, 0)),
            pl.BlockSpec((d, d), lambda i: (0, 0)),
            pl.BlockSpec((d, d), lambda i: (0, 0)),
            pl.BlockSpec((1, d), lambda i: (0, 0)),
        ],
        out_specs=pl.BlockSpec((blk, d), lambda i: (i, 0)),
        out_shape=jax.ShapeDtypeStruct((n_nodes, d), jnp.float32),
    )(x, parts, cnts, wx, wn, b.reshape(1, d))


def kernel(x, edge_index, W0, b0, W1, b1, W2, b2):
    n_nodes, d = x.shape
    n_edges = edge_index.shape[1]
    e_tile = n_edges // (NC * NS)
    nchunk = -(-e_tile // (K * IB)) * IB
    pad = nchunk * K - e_tile

    lane = jnp.arange(pad, dtype=jnp.int32) % SINK
    src2 = edge_index[0].astype(jnp.int32).reshape(NC * NS, e_tile)
    dst2 = edge_index[1].astype(jnp.int32).reshape(NC * NS, e_tile)
    src_pad = jnp.broadcast_to(lane, (NC * NS, pad))
    dst_pad = jnp.broadcast_to(n_nodes + lane, (NC * NS, pad))
    src4 = jnp.concatenate([src2, src_pad], 1).reshape(NC, NS, nchunk, K)
    dst4 = jnp.concatenate([dst2, dst_pad], 1).reshape(NC, NS, nchunk, K)

    cnts = _sc_count(dst4, n_nodes, d, nchunk)
    for (w, b) in ((W0, b0), (W1, b1), (W2, b2)):
        parts = _sc_agg(x, src4, dst4, n_nodes, d, nchunk)
        x = _tc_layer(x, parts, cnts, w[:d], w[d:], b, n_nodes, d)
    return x
